# Initial kernel scaffold; baseline (speedup 1.0000x reference)
import functools

import jax
import jax.numpy as jnp
from jax import lax
from jax.experimental import pallas as pl
from jax.experimental.pallas import tpu as pltpu
from jax.experimental.pallas import tpu_sc as plsc

N = 10000
E = 320000
D = 128
H = 16
C = 47

NC = 2
NS = 16
NW = NC * NS
EPW = E // NW
BLK = 80
NBLK = EPW // BLK
NPAD = 10240
RPT = NPAD // NS

_mesh = plsc.VectorSubcoreMesh(core_axis_name="c", subcore_axis_name="s")


def _seg_sum_body(with_counts, *refs):
    if with_counts:
        (table, srcv, dstv, acc_out, cnt_out,
         stage, row_buf, src_idx, dst_idx, ones_buf, zc, acc_sh, cnt_sh, sem) = refs
    else:
        (table, srcv, dstv, acc_out,
         stage, row_buf, src_idx, dst_idx, acc_sh, sem) = refs
    cid = lax.axis_index("c")
    sid = lax.axis_index("s")
    wid = sid * NC + cid

    def zrow(i, carry):
        stage[i, :] = jnp.zeros((16,), jnp.float32)
        return carry
    lax.fori_loop(0, RPT, zrow, 0)
    pltpu.sync_copy(stage, acc_sh.at[pl.ds(sid * RPT, RPT)])
    if with_counts:
        def zrow1(i, carry):
            zc[pl.ds(i * 16, 16)] = jnp.zeros((16,), jnp.float32)
            return carry
        lax.fori_loop(0, RPT // 16, zrow1, 0)
        pltpu.sync_copy(zc, cnt_sh.at[pl.ds(sid * RPT, RPT)])
        def orow(i, carry):
            ones_buf[pl.ds(i * 16, 16)] = jnp.ones((16,), jnp.float32)
            return carry
        lax.fori_loop(0, BLK // 16, orow, 0)
    plsc.subcore_barrier()

    pltpu.sync_copy(srcv.at[wid], src_idx)
    pltpu.sync_copy(dstv.at[wid], dst_idx)

    def step(j, carry):
        pltpu.async_copy(table.at[src_idx.at[j]], row_buf, sem).wait()
        pltpu.sync_copy(row_buf, acc_sh.at[dst_idx.at[j]], add=True)
        if with_counts:
            pltpu.sync_copy(ones_buf, cnt_sh.at[dst_idx.at[j]], add=True)
        return carry
    lax.fori_loop(0, NBLK, step, 0)
    plsc.subcore_barrier()

    pltpu.sync_copy(acc_sh.at[pl.ds(sid * RPT, RPT)], stage)
    pltpu.sync_copy(stage, acc_out.at[cid, pl.ds(sid * RPT, RPT)])
    if with_counts:
        pltpu.sync_copy(cnt_sh.at[pl.ds(sid * RPT, RPT)], zc)
        pltpu.sync_copy(zc, cnt_out.at[cid, pl.ds(sid * RPT, RPT)])


def _make_seg_sum(with_counts):
    out_type = [jax.ShapeDtypeStruct((NC, NPAD, H), jnp.float32)]
    scratch = [
        pltpu.VMEM((RPT, H), jnp.float32),
        pltpu.VMEM((BLK, H), jnp.float32),
        pltpu.VMEM((NBLK, BLK), jnp.int32),
        pltpu.VMEM((NBLK, BLK), jnp.int32),
    ]
    if with_counts:
        out_type.append(jax.ShapeDtypeStruct((NC, NPAD), jnp.float32))
        scratch += [
            pltpu.VMEM((BLK,), jnp.float32),
            pltpu.VMEM((RPT,), jnp.float32),
        ]
    scratch.append(pltpu.VMEM_SHARED((NPAD, H), jnp.float32))
    if with_counts:
        scratch.append(pltpu.VMEM_SHARED((NPAD,), jnp.float32))
    scratch.append(pltpu.SemaphoreType.DMA)

    def body(table, srcv, dstv, *rest):
        if with_counts:
            acc_out, cnt_out = rest[0], rest[1]
            _seg_sum_body(True, table, srcv, dstv, acc_out, cnt_out, *rest[2:])
        else:
            _seg_sum_body(False, table, srcv, dstv, *rest)

    return pl.kernel(body, out_type=tuple(out_type), mesh=_mesh,
                     scratch_types=tuple(scratch))


_seg_sum_cnt = _make_seg_sum(True)
_seg_sum = _make_seg_sum(False)

_ROWS = 400
_GRID = N // _ROWS


def _proj_body(x_ref, wl_ref, wr_ref, b_ref, xp_ref, xr_ref):
    xb = x_ref[...]
    dn = (((1,), (1,)), ((), ()))
    xp_ref[...] = lax.dot_general(xb, wl_ref[...], dn,
                                  preferred_element_type=jnp.float32)
    xr_ref[...] = lax.dot_general(xb, wr_ref[...], dn,
                                  preferred_element_type=jnp.float32) + b_ref[...]


def _h_body(acc_ref, cnt_ref, xr_ref, h_ref):
    a = acc_ref[0] + acc_ref[1]
    c = jnp.maximum(cnt_ref[0] + cnt_ref[1], 1.0)
    h_ref[...] = jnp.maximum(a / c[:, None] + xr_ref[...], 0.0)


def _out_body(acc_ref, cnt_ref, h_ref, wl_ref, wr_ref, b_ref, o_ref):
    a = acc_ref[0] + acc_ref[1]
    c = jnp.maximum(cnt_ref[0] + cnt_ref[1], 1.0)
    m = a / c[:, None]
    dn = (((1,), (1,)), ((), ()))
    o_ref[...] = (lax.dot_general(m, wl_ref[...], dn,
                                  preferred_element_type=jnp.float32)
                  + lax.dot_general(h_ref[...], wr_ref[...], dn,
                                    preferred_element_type=jnp.float32)
                  + b_ref[...])


def kernel(x, edge_index, W1l, b1, W1r, W2l, b2, W2r):
    src = edge_index[0].reshape(NW, NBLK, BLK)
    dst = edge_index[1].reshape(NW, NBLK, BLK)

    xp, xr = pl.pallas_call(
        _proj_body,
        grid=(_GRID,),
        in_specs=[
            pl.BlockSpec((_ROWS, D), lambda i: (i, 0)),
            pl.BlockSpec((H, D), lambda i: (0, 0)),
            pl.BlockSpec((H, D), lambda i: (0, 0)),
            pl.BlockSpec((1, H), lambda i: (0, 0)),
        ],
        out_specs=[
            pl.BlockSpec((_ROWS, H), lambda i: (i, 0)),
            pl.BlockSpec((_ROWS, H), lambda i: (i, 0)),
        ],
        out_shape=[
            jax.ShapeDtypeStruct((N, H), jnp.float32),
            jax.ShapeDtypeStruct((N, H), jnp.float32),
        ],
    )(x, W1l, W1r, b1[None, :])

    acc1, cnt = _seg_sum_cnt(xp, src, dst)

    h = pl.pallas_call(
        _h_body,
        grid=(_GRID,),
        in_specs=[
            pl.BlockSpec((NC, _ROWS, H), lambda i: (0, i, 0)),
            pl.BlockSpec((NC, _ROWS), lambda i: (0, i)),
            pl.BlockSpec((_ROWS, H), lambda i: (i, 0)),
        ],
        out_specs=pl.BlockSpec((_ROWS, H), lambda i: (i, 0)),
        out_shape=jax.ShapeDtypeStruct((N, H), jnp.float32),
    )(acc1, cnt, xr)

    (acc2,) = _seg_sum(h, src, dst)

    out = pl.pallas_call(
        _out_body,
        grid=(_GRID,),
        in_specs=[
            pl.BlockSpec((NC, _ROWS, H), lambda i: (0, i, 0)),
            pl.BlockSpec((NC, _ROWS), lambda i: (0, i)),
            pl.BlockSpec((_ROWS, H), lambda i: (i, 0)),
            pl.BlockSpec((C, H), lambda i: (0, 0)),
            pl.BlockSpec((C, H), lambda i: (0, 0)),
            pl.BlockSpec((1, C), lambda i: (0, 0)),
        ],
        out_specs=pl.BlockSpec((_ROWS, C), lambda i: (i, 0)),
        out_shape=jax.ShapeDtypeStruct((N, C), jnp.float32),
    )(acc2, cnt, h, W2l, W2r, b2[None, :])

    return out


# speedup vs baseline: 10.5096x; 10.5096x over previous
import functools

import jax
import jax.numpy as jnp
from jax import lax
from jax.experimental import pallas as pl
from jax.experimental.pallas import tpu as pltpu
from jax.experimental.pallas import tpu_sc as plsc

N = 10000
E = 320000
D = 128
H = 16
C = 47

NC = 2
NS = 16
NW = NC * NS
EPW = E // NW
BLK = 80
NBLK = EPW // BLK
NPAD = 10240
RPT = NPAD // NS

_mesh = plsc.VectorSubcoreMesh(core_axis_name="c", subcore_axis_name="s")


def _seg_sum_body(with_counts, *refs):
    if with_counts:
        (table, srcv, dstv, acc_out, cnt_out,
         stage, row_buf, src_idx, dst_idx, ones_buf, zc, acc_sh, cnt_sh, sem) = refs
    else:
        (table, srcv, dstv, acc_out,
         stage, row_buf, src_idx, dst_idx, acc_sh, sem) = refs
    cid = lax.axis_index("c")
    sid = lax.axis_index("s")
    wid = sid * NC + cid

    def zrow(i, carry):
        stage[i, :] = jnp.zeros((16,), jnp.float32)
        return carry
    lax.fori_loop(0, RPT, zrow, 0)
    pltpu.sync_copy(stage, acc_sh.at[pl.ds(sid * RPT, RPT)])
    if with_counts:
        def zrow1(i, carry):
            zc[pl.ds(i * 16, 16)] = jnp.zeros((16,), jnp.float32)
            return carry
        lax.fori_loop(0, RPT // 16, zrow1, 0)
        pltpu.sync_copy(zc, cnt_sh.at[pl.ds(sid * RPT, RPT)])
        def orow(i, carry):
            ones_buf[pl.ds(i * 16, 16)] = jnp.ones((16,), jnp.float32)
            return carry
        lax.fori_loop(0, BLK // 16, orow, 0)
    plsc.subcore_barrier()

    pltpu.sync_copy(srcv.at[wid], src_idx)
    pltpu.sync_copy(dstv.at[wid], dst_idx)

    def step(j, carry):
        pltpu.async_copy(table.at[src_idx.at[j]], row_buf, sem).wait()
        pltpu.sync_copy(row_buf, acc_sh.at[dst_idx.at[j]], add=True)
        if with_counts:
            pltpu.sync_copy(ones_buf, cnt_sh.at[dst_idx.at[j]], add=True)
        return carry
    lax.fori_loop(0, NBLK, step, 0)
    plsc.subcore_barrier()

    pltpu.sync_copy(acc_sh.at[pl.ds(sid * RPT, RPT)], stage)
    pltpu.sync_copy(stage, acc_out.at[cid, pl.ds(sid * RPT, RPT)])
    if with_counts:
        pltpu.sync_copy(cnt_sh.at[pl.ds(sid * RPT, RPT)], zc)
        pltpu.sync_copy(zc, cnt_out.at[cid, pl.ds(sid * RPT, RPT)])


def _make_seg_sum(with_counts):
    out_type = [jax.ShapeDtypeStruct((NC, NPAD, H), jnp.float32)]
    scratch = [
        pltpu.VMEM((RPT, H), jnp.float32),
        pltpu.VMEM((BLK, H), jnp.float32),
        pltpu.VMEM((NBLK, BLK), jnp.int32),
        pltpu.VMEM((NBLK, BLK), jnp.int32),
    ]
    if with_counts:
        out_type.append(jax.ShapeDtypeStruct((NC, NPAD), jnp.float32))
        scratch += [
            pltpu.VMEM((BLK,), jnp.float32),
            pltpu.VMEM((RPT,), jnp.float32),
        ]
    scratch.append(pltpu.VMEM_SHARED((NPAD, H), jnp.float32))
    if with_counts:
        scratch.append(pltpu.VMEM_SHARED((NPAD,), jnp.float32))
    scratch.append(pltpu.SemaphoreType.DMA)

    def body(table, srcv, dstv, *rest):
        if with_counts:
            acc_out, cnt_out = rest[0], rest[1]
            _seg_sum_body(True, table, srcv, dstv, acc_out, cnt_out, *rest[2:])
        else:
            _seg_sum_body(False, table, srcv, dstv, *rest)

    return pl.kernel(body, out_type=tuple(out_type), mesh=_mesh,
                     scratch_types=tuple(scratch),
                     compiler_params=pltpu.CompilerParams(
                         use_tc_tiling_on_sc=False))


_seg_sum_cnt = _make_seg_sum(True)
_seg_sum = _make_seg_sum(False)

_ROWS = 400
_GRID = N // _ROWS


def _proj_body(x_ref, wl_ref, wr_ref, b_ref, xp_ref, xr_ref):
    xb = x_ref[...]
    dn = (((1,), (1,)), ((), ()))
    xp_ref[...] = lax.dot_general(xb, wl_ref[...], dn,
                                  preferred_element_type=jnp.float32)
    xr_ref[...] = lax.dot_general(xb, wr_ref[...], dn,
                                  preferred_element_type=jnp.float32) + b_ref[...]


def _h_body(acc_ref, cnt_ref, xr_ref, h_ref):
    a = acc_ref[0] + acc_ref[1]
    c = jnp.maximum(cnt_ref[0] + cnt_ref[1], 1.0)
    h_ref[...] = jnp.maximum(a / c + xr_ref[...], 0.0)


def _out_body(acc_ref, cnt_ref, h_ref, wl_ref, wr_ref, b_ref, o_ref):
    a = acc_ref[0] + acc_ref[1]
    c = jnp.maximum(cnt_ref[0] + cnt_ref[1], 1.0)
    m = a / c
    dn = (((1,), (1,)), ((), ()))
    o_ref[...] = (lax.dot_general(m, wl_ref[...], dn,
                                  preferred_element_type=jnp.float32)
                  + lax.dot_general(h_ref[...], wr_ref[...], dn,
                                    preferred_element_type=jnp.float32)
                  + b_ref[...])


def kernel(x, edge_index, W1l, b1, W1r, W2l, b2, W2r):
    src = edge_index[0].reshape(NW, NBLK, BLK)
    dst = edge_index[1].reshape(NW, NBLK, BLK)

    xp, xr = pl.pallas_call(
        _proj_body,
        grid=(_GRID,),
        in_specs=[
            pl.BlockSpec((_ROWS, D), lambda i: (i, 0)),
            pl.BlockSpec((H, D), lambda i: (0, 0)),
            pl.BlockSpec((H, D), lambda i: (0, 0)),
            pl.BlockSpec((1, H), lambda i: (0, 0)),
        ],
        out_specs=[
            pl.BlockSpec((_ROWS, H), lambda i: (i, 0)),
            pl.BlockSpec((_ROWS, H), lambda i: (i, 0)),
        ],
        out_shape=[
            jax.ShapeDtypeStruct((N, H), jnp.float32),
            jax.ShapeDtypeStruct((N, H), jnp.float32),
        ],
    )(x, W1l, W1r, b1[None, :])

    acc1, cnt = _seg_sum_cnt(xp, src, dst)
    cnt = cnt[..., None]

    h = pl.pallas_call(
        _h_body,
        grid=(_GRID,),
        in_specs=[
            pl.BlockSpec((NC, _ROWS, H), lambda i: (0, i, 0)),
            pl.BlockSpec((NC, _ROWS, 1), lambda i: (0, i, 0)),
            pl.BlockSpec((_ROWS, H), lambda i: (i, 0)),
        ],
        out_specs=pl.BlockSpec((_ROWS, H), lambda i: (i, 0)),
        out_shape=jax.ShapeDtypeStruct((N, H), jnp.float32),
    )(acc1, cnt, xr)

    (acc2,) = _seg_sum(h, src, dst)

    out = pl.pallas_call(
        _out_body,
        grid=(_GRID,),
        in_specs=[
            pl.BlockSpec((NC, _ROWS, H), lambda i: (0, i, 0)),
            pl.BlockSpec((NC, _ROWS, 1), lambda i: (0, i, 0)),
            pl.BlockSpec((_ROWS, H), lambda i: (i, 0)),
            pl.BlockSpec((C, H), lambda i: (0, 0)),
            pl.BlockSpec((C, H), lambda i: (0, 0)),
            pl.BlockSpec((1, C), lambda i: (0, 0)),
        ],
        out_specs=pl.BlockSpec((_ROWS, C), lambda i: (i, 0)),
        out_shape=jax.ShapeDtypeStruct((N, C), jnp.float32),
    )(acc2, cnt, h, W2l, W2r, b2[None, :])

    return out


# speedup vs baseline: 14.1064x; 1.3422x over previous
import functools

import jax
import jax.numpy as jnp
from jax import lax
from jax.experimental import pallas as pl
from jax.experimental.pallas import tpu as pltpu
from jax.experimental.pallas import tpu_sc as plsc

N = 10000
E = 320000
D = 128
H = 16
C = 47

NC = 2
NS = 16
NW = NC * NS
EPW = E // NW
BLK = 128
NBLK = 80
EPT = NBLK * BLK
NBUF = 4
NPAD = 10240
DUMMY = NPAD - 2
RPT = NPAD // NS

_mesh = plsc.VectorSubcoreMesh(core_axis_name="c", subcore_axis_name="s")


def _seg_sum_body(with_counts, *refs):
    if with_counts:
        (table, srcv, dstv, acc_out, cnt_out,
         stage, row_bufs, src_idx, dst_idx, ones_buf, zc,
         acc_sh, cnt_sh, sems) = refs
    else:
        (table, srcv, dstv, acc_out,
         stage, row_bufs, src_idx, dst_idx, acc_sh, sems) = refs
    cid = lax.axis_index("c")
    sid = lax.axis_index("s")
    wid = sid * NC + cid

    def zrow(i, carry):
        stage[i, :] = jnp.zeros((16,), jnp.float32)
        return carry
    lax.fori_loop(0, RPT, zrow, 0)
    pltpu.sync_copy(stage, acc_sh.at[pl.ds(sid * RPT, RPT)])
    if with_counts:
        def zrow1(i, carry):
            zc[pl.ds(i * 16, 16)] = jnp.zeros((16,), jnp.float32)
            return carry
        lax.fori_loop(0, RPT // 16, zrow1, 0)
        pltpu.sync_copy(zc, cnt_sh.at[pl.ds(sid * RPT, RPT)])
        def orow(i, carry):
            ones_buf[pl.ds(i * 16, 16)] = jnp.ones((16,), jnp.float32)
            return carry
        lax.fori_loop(0, BLK // 16, orow, 0)
    plsc.subcore_barrier()

    pltpu.sync_copy(srcv.at[wid], src_idx)
    pltpu.sync_copy(dstv.at[wid], dst_idx)

    for b in range(NBUF):
        pltpu.async_copy(table.at[src_idx.at[b]], row_bufs[b], sems[b])

    def group(g, carry):
        for b in range(NBUF):
            j = g * NBUF + b
            pltpu.make_async_copy(
                table.at[src_idx.at[j]], row_bufs[b], sems[b]).wait()
            pltpu.sync_copy(row_bufs[b], acc_sh.at[dst_idx.at[j]], add=True)
            if with_counts:
                pltpu.sync_copy(ones_buf, cnt_sh.at[dst_idx.at[j]], add=True)

            @pl.when(j + NBUF < NBLK)
            def _():
                pltpu.async_copy(
                    table.at[src_idx.at[j + NBUF]], row_bufs[b], sems[b])
        return carry
    lax.fori_loop(0, NBLK // NBUF, group, 0)
    plsc.subcore_barrier()

    pltpu.sync_copy(acc_sh.at[pl.ds(sid * RPT, RPT)], stage)
    pltpu.sync_copy(stage, acc_out.at[cid, pl.ds(sid * RPT, RPT)])
    if with_counts:
        pltpu.sync_copy(cnt_sh.at[pl.ds(sid * RPT, RPT)], zc)
        pltpu.sync_copy(zc, cnt_out.at[cid, pl.ds(sid * RPT, RPT)])


def _make_seg_sum(with_counts):
    out_type = [jax.ShapeDtypeStruct((NC, NPAD, H), jnp.float32)]
    scratch = [
        pltpu.VMEM((RPT, H), jnp.float32),
        [pltpu.VMEM((BLK, H), jnp.float32) for _ in range(NBUF)],
        pltpu.VMEM((NBLK, BLK), jnp.int32),
        pltpu.VMEM((NBLK, BLK), jnp.int32),
    ]
    if with_counts:
        out_type.append(jax.ShapeDtypeStruct((NC, NPAD), jnp.float32))
        scratch += [
            pltpu.VMEM((BLK,), jnp.float32),
            pltpu.VMEM((RPT,), jnp.float32),
        ]
    scratch.append(pltpu.VMEM_SHARED((NPAD, H), jnp.float32))
    if with_counts:
        scratch.append(pltpu.VMEM_SHARED((NPAD,), jnp.float32))
    scratch.append([pltpu.SemaphoreType.DMA for _ in range(NBUF)])

    def body(table, srcv, dstv, *rest):
        if with_counts:
            acc_out, cnt_out = rest[0], rest[1]
            _seg_sum_body(True, table, srcv, dstv, acc_out, cnt_out, *rest[2:])
        else:
            _seg_sum_body(False, table, srcv, dstv, *rest)

    return pl.kernel(body, out_type=tuple(out_type), mesh=_mesh,
                     scratch_types=tuple(scratch),
                     compiler_params=pltpu.CompilerParams(
                         use_tc_tiling_on_sc=False))


_seg_sum_cnt = _make_seg_sum(True)
_seg_sum = _make_seg_sum(False)

_ROWS = 400
_GRID = N // _ROWS


def _proj_body(x_ref, wl_ref, wr_ref, b_ref, xp_ref, xr_ref):
    xb = x_ref[...]
    dn = (((1,), (1,)), ((), ()))
    xp_ref[...] = lax.dot_general(xb, wl_ref[...], dn,
                                  preferred_element_type=jnp.float32)
    xr_ref[...] = lax.dot_general(xb, wr_ref[...], dn,
                                  preferred_element_type=jnp.float32) + b_ref[...]


def _h_body(acc_ref, cnt_ref, xr_ref, h_ref):
    a = acc_ref[0] + acc_ref[1]
    c = jnp.maximum(cnt_ref[0] + cnt_ref[1], 1.0)
    h_ref[...] = jnp.maximum(a / c + xr_ref[...], 0.0)


def _out_body(acc_ref, cnt_ref, h_ref, wl_ref, wr_ref, b_ref, o_ref):
    a = acc_ref[0] + acc_ref[1]
    c = jnp.maximum(cnt_ref[0] + cnt_ref[1], 1.0)
    m = a / c
    dn = (((1,), (1,)), ((), ()))
    o_ref[...] = (lax.dot_general(m, wl_ref[...], dn,
                                  preferred_element_type=jnp.float32)
                  + lax.dot_general(h_ref[...], wr_ref[...], dn,
                                    preferred_element_type=jnp.float32)
                  + b_ref[...])


def kernel(x, edge_index, W1l, b1, W1r, W2l, b2, W2r):
    pad = EPT - EPW
    src = jnp.concatenate(
        [edge_index[0].reshape(NW, EPW),
         jnp.zeros((NW, pad), jnp.int32)], axis=1).reshape(NW, NBLK, BLK)
    dst = jnp.concatenate(
        [edge_index[1].reshape(NW, EPW),
         jnp.full((NW, pad), DUMMY, jnp.int32)], axis=1).reshape(NW, NBLK, BLK)

    xp, xr = pl.pallas_call(
        _proj_body,
        grid=(_GRID,),
        in_specs=[
            pl.BlockSpec((_ROWS, D), lambda i: (i, 0)),
            pl.BlockSpec((H, D), lambda i: (0, 0)),
            pl.BlockSpec((H, D), lambda i: (0, 0)),
            pl.BlockSpec((1, H), lambda i: (0, 0)),
        ],
        out_specs=[
            pl.BlockSpec((_ROWS, H), lambda i: (i, 0)),
            pl.BlockSpec((_ROWS, H), lambda i: (i, 0)),
        ],
        out_shape=[
            jax.ShapeDtypeStruct((N, H), jnp.float32),
            jax.ShapeDtypeStruct((N, H), jnp.float32),
        ],
    )(x, W1l, W1r, b1[None, :])

    acc1, cnt = _seg_sum_cnt(xp, src, dst)
    cnt = cnt[..., None]

    h = pl.pallas_call(
        _h_body,
        grid=(_GRID,),
        in_specs=[
            pl.BlockSpec((NC, _ROWS, H), lambda i: (0, i, 0)),
            pl.BlockSpec((NC, _ROWS, 1), lambda i: (0, i, 0)),
            pl.BlockSpec((_ROWS, H), lambda i: (i, 0)),
        ],
        out_specs=pl.BlockSpec((_ROWS, H), lambda i: (i, 0)),
        out_shape=jax.ShapeDtypeStruct((N, H), jnp.float32),
    )(acc1, cnt, xr)

    (acc2,) = _seg_sum(h, src, dst)

    out = pl.pallas_call(
        _out_body,
        grid=(_GRID,),
        in_specs=[
            pl.BlockSpec((NC, _ROWS, H), lambda i: (0, i, 0)),
            pl.BlockSpec((NC, _ROWS, 1), lambda i: (0, i, 0)),
            pl.BlockSpec((_ROWS, H), lambda i: (i, 0)),
            pl.BlockSpec((C, H), lambda i: (0, 0)),
            pl.BlockSpec((C, H), lambda i: (0, 0)),
            pl.BlockSpec((1, C), lambda i: (0, 0)),
        ],
        out_specs=pl.BlockSpec((_ROWS, C), lambda i: (i, 0)),
        out_shape=jax.ShapeDtypeStruct((N, C), jnp.float32),
    )(acc2, cnt, h, W2l, W2r, b2[None, :])

    return out


# speedup vs baseline: 14.4764x; 1.0262x over previous
import functools

import jax
import jax.numpy as jnp
from jax import lax
from jax.experimental import pallas as pl
from jax.experimental.pallas import tpu as pltpu
from jax.experimental.pallas import tpu_sc as plsc

N = 10000
E = 320000
D = 128
H = 16
C = 47

NC = 2
NS = 16
NW = NC * NS
EPW = E // NW
BLK = 128
NBLK = 80
EPT = NBLK * BLK
NBUF = 8
LEAD = 4
NPAD = 10240
DUMMY = NPAD - 2
RPT = NPAD // NS

_mesh = plsc.VectorSubcoreMesh(core_axis_name="c", subcore_axis_name="s")


def _seg_sum_body(with_counts, *refs):
    if with_counts:
        (table, srcv, dstv, acc_out, cnt_out,
         stage, row_bufs, src_idx, dst_idx, ones_buf, zc,
         acc_sh, cnt_sh, sems) = refs
    else:
        (table, srcv, dstv, acc_out,
         stage, row_bufs, src_idx, dst_idx, acc_sh, sems) = refs
    cid = lax.axis_index("c")
    sid = lax.axis_index("s")
    wid = sid * NC + cid

    def zrow(i, carry):
        stage[i, :] = jnp.zeros((16,), jnp.float32)
        return carry
    lax.fori_loop(0, RPT, zrow, 0)
    pltpu.sync_copy(stage, acc_sh.at[pl.ds(sid * RPT, RPT)])
    if with_counts:
        def zrow1(i, carry):
            zc[pl.ds(i * 16, 16)] = jnp.zeros((16,), jnp.float32)
            return carry
        lax.fori_loop(0, RPT // 16, zrow1, 0)
        pltpu.sync_copy(zc, cnt_sh.at[pl.ds(sid * RPT, RPT)])
        def orow(i, carry):
            ones_buf[pl.ds(i * 16, 16)] = jnp.ones((16,), jnp.float32)
            return carry
        lax.fori_loop(0, BLK // 16, orow, 0)
    plsc.subcore_barrier()

    pltpu.sync_copy(srcv.at[wid], src_idx)
    pltpu.sync_copy(dstv.at[wid], dst_idx)

    gsems, ssems, csems = sems
    for b in range(NBUF):
        pltpu.async_copy(table.at[src_idx.at[b]], row_bufs[b], gsems[b])

    def group(g, carry):
        for b in range(NBUF):
            j = g * NBUF + b
            pltpu.make_async_copy(
                table.at[src_idx.at[j]], row_bufs[b], gsems[b]).wait()
            pltpu.async_copy(
                row_bufs[b], acc_sh.at[dst_idx.at[j]], ssems[b], add=True)
            if with_counts:
                pltpu.async_copy(
                    ones_buf, cnt_sh.at[dst_idx.at[j]], csems[b], add=True)

            bp = (b - LEAD) % NBUF

            @pl.when(jnp.logical_and(j >= LEAD, j + LEAD < NBLK))
            def _():
                jp = j - LEAD
                pltpu.make_async_copy(
                    row_bufs[bp], acc_sh.at[dst_idx.at[jp]], ssems[bp]).wait()
                if with_counts:
                    pltpu.make_async_copy(
                        ones_buf, cnt_sh.at[dst_idx.at[jp]], csems[bp]).wait()
                pltpu.async_copy(
                    table.at[src_idx.at[j + LEAD]], row_bufs[bp], gsems[bp])
        return carry
    lax.fori_loop(0, NBLK // NBUF, group, 0)

    for j in range(NBLK - NBUF, NBLK):
        b = j % NBUF
        pltpu.make_async_copy(
            row_bufs[b], acc_sh.at[dst_idx.at[j]], ssems[b]).wait()
        if with_counts:
            pltpu.make_async_copy(
                ones_buf, cnt_sh.at[dst_idx.at[j]], csems[b]).wait()
    plsc.subcore_barrier()

    pltpu.sync_copy(acc_sh.at[pl.ds(sid * RPT, RPT)], stage)
    pltpu.sync_copy(stage, acc_out.at[cid, pl.ds(sid * RPT, RPT)])
    if with_counts:
        pltpu.sync_copy(cnt_sh.at[pl.ds(sid * RPT, RPT)], zc)
        pltpu.sync_copy(zc, cnt_out.at[cid, pl.ds(sid * RPT, RPT)])


def _make_seg_sum(with_counts):
    out_type = [jax.ShapeDtypeStruct((NC, NPAD, H), jnp.float32)]
    scratch = [
        pltpu.VMEM((RPT, H), jnp.float32),
        [pltpu.VMEM((BLK, H), jnp.float32) for _ in range(NBUF)],
        pltpu.VMEM((NBLK, BLK), jnp.int32),
        pltpu.VMEM((NBLK, BLK), jnp.int32),
    ]
    if with_counts:
        out_type.append(jax.ShapeDtypeStruct((NC, NPAD), jnp.float32))
        scratch += [
            pltpu.VMEM((BLK,), jnp.float32),
            pltpu.VMEM((RPT,), jnp.float32),
        ]
    scratch.append(pltpu.VMEM_SHARED((NPAD, H), jnp.float32))
    if with_counts:
        scratch.append(pltpu.VMEM_SHARED((NPAD,), jnp.float32))
    scratch.append([[pltpu.SemaphoreType.DMA for _ in range(NBUF)]
                    for _ in range(3)])

    def body(table, srcv, dstv, *rest):
        if with_counts:
            acc_out, cnt_out = rest[0], rest[1]
            _seg_sum_body(True, table, srcv, dstv, acc_out, cnt_out, *rest[2:])
        else:
            _seg_sum_body(False, table, srcv, dstv, *rest)

    return pl.kernel(body, out_type=tuple(out_type), mesh=_mesh,
                     scratch_types=tuple(scratch),
                     compiler_params=pltpu.CompilerParams(
                         use_tc_tiling_on_sc=False))


_seg_sum_cnt = _make_seg_sum(True)
_seg_sum = _make_seg_sum(False)

_ROWS = 400
_GRID = N // _ROWS


def _proj_body(x_ref, wl_ref, wr_ref, b_ref, xp_ref, xr_ref):
    xb = x_ref[...]
    dn = (((1,), (1,)), ((), ()))
    xp_ref[...] = lax.dot_general(xb, wl_ref[...], dn,
                                  preferred_element_type=jnp.float32)
    xr_ref[...] = lax.dot_general(xb, wr_ref[...], dn,
                                  preferred_element_type=jnp.float32) + b_ref[...]


def _h_body(acc_ref, cnt_ref, xr_ref, h_ref):
    a = acc_ref[0] + acc_ref[1]
    c = jnp.maximum(cnt_ref[0] + cnt_ref[1], 1.0)
    h_ref[...] = jnp.maximum(a / c + xr_ref[...], 0.0)


def _out_body(acc_ref, cnt_ref, h_ref, wl_ref, wr_ref, b_ref, o_ref):
    a = acc_ref[0] + acc_ref[1]
    c = jnp.maximum(cnt_ref[0] + cnt_ref[1], 1.0)
    m = a / c
    dn = (((1,), (1,)), ((), ()))
    o_ref[...] = (lax.dot_general(m, wl_ref[...], dn,
                                  preferred_element_type=jnp.float32)
                  + lax.dot_general(h_ref[...], wr_ref[...], dn,
                                    preferred_element_type=jnp.float32)
                  + b_ref[...])


def kernel(x, edge_index, W1l, b1, W1r, W2l, b2, W2r):
    pad = EPT - EPW
    src = jnp.concatenate(
        [edge_index[0].reshape(NW, EPW),
         jnp.zeros((NW, pad), jnp.int32)], axis=1).reshape(NW, NBLK, BLK)
    dst = jnp.concatenate(
        [edge_index[1].reshape(NW, EPW),
         jnp.full((NW, pad), DUMMY, jnp.int32)], axis=1).reshape(NW, NBLK, BLK)

    xp, xr = pl.pallas_call(
        _proj_body,
        grid=(_GRID,),
        in_specs=[
            pl.BlockSpec((_ROWS, D), lambda i: (i, 0)),
            pl.BlockSpec((H, D), lambda i: (0, 0)),
            pl.BlockSpec((H, D), lambda i: (0, 0)),
            pl.BlockSpec((1, H), lambda i: (0, 0)),
        ],
        out_specs=[
            pl.BlockSpec((_ROWS, H), lambda i: (i, 0)),
            pl.BlockSpec((_ROWS, H), lambda i: (i, 0)),
        ],
        out_shape=[
            jax.ShapeDtypeStruct((N, H), jnp.float32),
            jax.ShapeDtypeStruct((N, H), jnp.float32),
        ],
    )(x, W1l, W1r, b1[None, :])

    acc1, cnt = _seg_sum_cnt(xp, src, dst)
    cnt = cnt[..., None]

    h = pl.pallas_call(
        _h_body,
        grid=(_GRID,),
        in_specs=[
            pl.BlockSpec((NC, _ROWS, H), lambda i: (0, i, 0)),
            pl.BlockSpec((NC, _ROWS, 1), lambda i: (0, i, 0)),
            pl.BlockSpec((_ROWS, H), lambda i: (i, 0)),
        ],
        out_specs=pl.BlockSpec((_ROWS, H), lambda i: (i, 0)),
        out_shape=jax.ShapeDtypeStruct((N, H), jnp.float32),
    )(acc1, cnt, xr)

    (acc2,) = _seg_sum(h, src, dst)

    out = pl.pallas_call(
        _out_body,
        grid=(_GRID,),
        in_specs=[
            pl.BlockSpec((NC, _ROWS, H), lambda i: (0, i, 0)),
            pl.BlockSpec((NC, _ROWS, 1), lambda i: (0, i, 0)),
            pl.BlockSpec((_ROWS, H), lambda i: (i, 0)),
            pl.BlockSpec((C, H), lambda i: (0, 0)),
            pl.BlockSpec((C, H), lambda i: (0, 0)),
            pl.BlockSpec((1, C), lambda i: (0, 0)),
        ],
        out_specs=pl.BlockSpec((_ROWS, C), lambda i: (i, 0)),
        out_shape=jax.ShapeDtypeStruct((N, C), jnp.float32),
    )(acc2, cnt, h, W2l, W2r, b2[None, :])

    return out


# speedup vs baseline: 19.7223x; 1.3624x over previous
import functools

import jax
import jax.numpy as jnp
from jax import lax
from jax.experimental import pallas as pl
from jax.experimental.pallas import tpu as pltpu
from jax.experimental.pallas import tpu_sc as plsc

N = 10000
E = 320000
D = 128
H = 16
C = 47

NC = 2
NS = 16
NW = NC * NS
EPW = E // NW
BLK = 128
NBLK = 80
EPT = NBLK * BLK
NBUF = 8
LEAD = 4
NPAD = 10240
DUMMY = NPAD - 2
RPT = NPAD // NS

_mesh = plsc.VectorSubcoreMesh(core_axis_name="c", subcore_axis_name="s")


def _seg_sum_body(with_counts, *refs):
    if with_counts:
        (table, srcv, dstv, acc_out, cnt_out,
         stage, row_bufs, src_idx, dst_idx, ones_buf, zc,
         table_sh, acc_sh, cnt_sh, sems) = refs
    else:
        (table, srcv, dstv, acc_out,
         stage, row_bufs, src_idx, dst_idx, table_sh, acc_sh, sems) = refs
    cid = lax.axis_index("c")
    sid = lax.axis_index("s")
    wid = sid * NC + cid

    tpt = N // NS
    pltpu.sync_copy(table.at[pl.ds(sid * tpt, tpt)], stage.at[pl.ds(0, tpt)])
    pltpu.sync_copy(stage.at[pl.ds(0, tpt)], table_sh.at[pl.ds(sid * tpt, tpt)])

    def zrow(i, carry):
        stage[i, :] = jnp.zeros((16,), jnp.float32)
        return carry
    lax.fori_loop(0, RPT, zrow, 0)
    pltpu.sync_copy(stage, acc_sh.at[pl.ds(sid * RPT, RPT)])
    if with_counts:
        def zrow1(i, carry):
            zc[pl.ds(i * 16, 16)] = jnp.zeros((16,), jnp.float32)
            return carry
        lax.fori_loop(0, RPT // 16, zrow1, 0)
        pltpu.sync_copy(zc, cnt_sh.at[pl.ds(sid * RPT, RPT)])
        def orow(i, carry):
            ones_buf[pl.ds(i * 16, 16)] = jnp.ones((16,), jnp.float32)
            return carry
        lax.fori_loop(0, BLK // 16, orow, 0)
    plsc.subcore_barrier()

    pltpu.sync_copy(srcv.at[wid], src_idx)
    pltpu.sync_copy(dstv.at[wid], dst_idx)

    gsems, ssems, csems = sems
    for b in range(NBUF):
        pltpu.async_copy(table_sh.at[src_idx.at[b]], row_bufs[b], gsems[b])

    def group(g, carry):
        for b in range(NBUF):
            j = g * NBUF + b
            pltpu.make_async_copy(
                table_sh.at[src_idx.at[j]], row_bufs[b], gsems[b]).wait()
            pltpu.async_copy(
                row_bufs[b], acc_sh.at[dst_idx.at[j]], ssems[b], add=True)
            if with_counts:
                pltpu.async_copy(
                    ones_buf, cnt_sh.at[dst_idx.at[j]], csems[b], add=True)

            bp = (b - LEAD) % NBUF

            @pl.when(jnp.logical_and(j >= LEAD, j + LEAD < NBLK))
            def _():
                jp = j - LEAD
                pltpu.make_async_copy(
                    row_bufs[bp], acc_sh.at[dst_idx.at[jp]], ssems[bp]).wait()
                if with_counts:
                    pltpu.make_async_copy(
                        ones_buf, cnt_sh.at[dst_idx.at[jp]], csems[bp]).wait()
                pltpu.async_copy(
                    table_sh.at[src_idx.at[j + LEAD]], row_bufs[bp], gsems[bp])
        return carry
    lax.fori_loop(0, NBLK // NBUF, group, 0)

    for j in range(NBLK - NBUF, NBLK):
        b = j % NBUF
        pltpu.make_async_copy(
            row_bufs[b], acc_sh.at[dst_idx.at[j]], ssems[b]).wait()
        if with_counts:
            pltpu.make_async_copy(
                ones_buf, cnt_sh.at[dst_idx.at[j]], csems[b]).wait()
    plsc.subcore_barrier()

    pltpu.sync_copy(acc_sh.at[pl.ds(sid * RPT, RPT)], stage)
    pltpu.sync_copy(stage, acc_out.at[cid, pl.ds(sid * RPT, RPT)])
    if with_counts:
        pltpu.sync_copy(cnt_sh.at[pl.ds(sid * RPT, RPT)], zc)
        pltpu.sync_copy(zc, cnt_out.at[cid, pl.ds(sid * RPT, RPT)])


def _make_seg_sum(with_counts):
    out_type = [jax.ShapeDtypeStruct((NC, NPAD, H), jnp.float32)]
    scratch = [
        pltpu.VMEM((RPT, H), jnp.float32),
        [pltpu.VMEM((BLK, H), jnp.float32) for _ in range(NBUF)],
        pltpu.VMEM((NBLK, BLK), jnp.int32),
        pltpu.VMEM((NBLK, BLK), jnp.int32),
    ]
    if with_counts:
        out_type.append(jax.ShapeDtypeStruct((NC, NPAD), jnp.float32))
        scratch += [
            pltpu.VMEM((BLK,), jnp.float32),
            pltpu.VMEM((RPT,), jnp.float32),
        ]
    scratch.append(pltpu.VMEM_SHARED((N, H), jnp.float32))
    scratch.append(pltpu.VMEM_SHARED((NPAD, H), jnp.float32))
    if with_counts:
        scratch.append(pltpu.VMEM_SHARED((NPAD,), jnp.float32))
    scratch.append([[pltpu.SemaphoreType.DMA for _ in range(NBUF)]
                    for _ in range(3)])

    def body(table, srcv, dstv, *rest):
        if with_counts:
            acc_out, cnt_out = rest[0], rest[1]
            _seg_sum_body(True, table, srcv, dstv, acc_out, cnt_out, *rest[2:])
        else:
            _seg_sum_body(False, table, srcv, dstv, *rest)

    return pl.kernel(body, out_type=tuple(out_type), mesh=_mesh,
                     scratch_types=tuple(scratch),
                     compiler_params=pltpu.CompilerParams(
                         use_tc_tiling_on_sc=False))


_seg_sum_cnt = _make_seg_sum(True)
_seg_sum = _make_seg_sum(False)

_ROWS = 400
_GRID = N // _ROWS


def _proj_body(x_ref, wl_ref, wr_ref, b_ref, xp_ref, xr_ref):
    xb = x_ref[...]
    dn = (((1,), (1,)), ((), ()))
    xp_ref[...] = lax.dot_general(xb, wl_ref[...], dn,
                                  preferred_element_type=jnp.float32)
    xr_ref[...] = lax.dot_general(xb, wr_ref[...], dn,
                                  preferred_element_type=jnp.float32) + b_ref[...]


def _h_body(acc_ref, cnt_ref, xr_ref, h_ref):
    a = acc_ref[0] + acc_ref[1]
    c = jnp.maximum(cnt_ref[0] + cnt_ref[1], 1.0)
    h_ref[...] = jnp.maximum(a / c + xr_ref[...], 0.0)


def _out_body(acc_ref, cnt_ref, h_ref, wl_ref, wr_ref, b_ref, o_ref):
    a = acc_ref[0] + acc_ref[1]
    c = jnp.maximum(cnt_ref[0] + cnt_ref[1], 1.0)
    m = a / c
    dn = (((1,), (1,)), ((), ()))
    o_ref[...] = (lax.dot_general(m, wl_ref[...], dn,
                                  preferred_element_type=jnp.float32)
                  + lax.dot_general(h_ref[...], wr_ref[...], dn,
                                    preferred_element_type=jnp.float32)
                  + b_ref[...])


def kernel(x, edge_index, W1l, b1, W1r, W2l, b2, W2r):
    pad = EPT - EPW
    src = jnp.concatenate(
        [edge_index[0].reshape(NW, EPW),
         jnp.zeros((NW, pad), jnp.int32)], axis=1).reshape(NW, NBLK, BLK)
    dst = jnp.concatenate(
        [edge_index[1].reshape(NW, EPW),
         jnp.full((NW, pad), DUMMY, jnp.int32)], axis=1).reshape(NW, NBLK, BLK)

    xp, xr = pl.pallas_call(
        _proj_body,
        grid=(_GRID,),
        in_specs=[
            pl.BlockSpec((_ROWS, D), lambda i: (i, 0)),
            pl.BlockSpec((H, D), lambda i: (0, 0)),
            pl.BlockSpec((H, D), lambda i: (0, 0)),
            pl.BlockSpec((1, H), lambda i: (0, 0)),
        ],
        out_specs=[
            pl.BlockSpec((_ROWS, H), lambda i: (i, 0)),
            pl.BlockSpec((_ROWS, H), lambda i: (i, 0)),
        ],
        out_shape=[
            jax.ShapeDtypeStruct((N, H), jnp.float32),
            jax.ShapeDtypeStruct((N, H), jnp.float32),
        ],
    )(x, W1l, W1r, b1[None, :])

    acc1, cnt = _seg_sum_cnt(xp, src, dst)
    cnt = cnt[..., None]

    h = pl.pallas_call(
        _h_body,
        grid=(_GRID,),
        in_specs=[
            pl.BlockSpec((NC, _ROWS, H), lambda i: (0, i, 0)),
            pl.BlockSpec((NC, _ROWS, 1), lambda i: (0, i, 0)),
            pl.BlockSpec((_ROWS, H), lambda i: (i, 0)),
        ],
        out_specs=pl.BlockSpec((_ROWS, H), lambda i: (i, 0)),
        out_shape=jax.ShapeDtypeStruct((N, H), jnp.float32),
    )(acc1, cnt, xr)

    (acc2,) = _seg_sum(h, src, dst)

    out = pl.pallas_call(
        _out_body,
        grid=(_GRID,),
        in_specs=[
            pl.BlockSpec((NC, _ROWS, H), lambda i: (0, i, 0)),
            pl.BlockSpec((NC, _ROWS, 1), lambda i: (0, i, 0)),
            pl.BlockSpec((_ROWS, H), lambda i: (i, 0)),
            pl.BlockSpec((C, H), lambda i: (0, 0)),
            pl.BlockSpec((C, H), lambda i: (0, 0)),
            pl.BlockSpec((1, C), lambda i: (0, 0)),
        ],
        out_specs=pl.BlockSpec((_ROWS, C), lambda i: (i, 0)),
        out_shape=jax.ShapeDtypeStruct((N, C), jnp.float32),
    )(acc2, cnt, h, W2l, W2r, b2[None, :])

    return out


# speedup vs baseline: 20.7557x; 1.0524x over previous
import jax
import jax.numpy as jnp
from jax import lax
from jax.experimental import pallas as pl
from jax.experimental.pallas import tpu as pltpu
from jax.experimental.pallas import tpu_sc as plsc

N = 10000
E = 320000
D = 128
H = 16
C = 47

NC = 2
NS = 16
NW = NC * NS
BLK = 128
NBUF = 8
LEAD = 4
NBLK1 = 160
NBLK2 = 80
NPAD = 10240
DUMMY = NPAD - 2
RPT = NPAD // NS
TPT = N // NS

_mesh = plsc.VectorSubcoreMesh(core_axis_name="c", subcore_axis_name="s")


def _ring_loop(table_sh, acc_sh, cnt_sh, src_idx, dst_idx, nblk,
               row_bufs, ones_buf, gsems, ssems, csems):
    for b in range(NBUF):
        pltpu.async_copy(table_sh.at[src_idx.at[b]], row_bufs[b], gsems[b])

    def group(g, carry):
        for b in range(NBUF):
            j = g * NBUF + b
            pltpu.make_async_copy(
                table_sh.at[src_idx.at[j]], row_bufs[b], gsems[b]).wait()
            pltpu.async_copy(
                row_bufs[b], acc_sh.at[dst_idx.at[j]], ssems[b], add=True)
            if cnt_sh is not None:
                pltpu.async_copy(
                    ones_buf, cnt_sh.at[dst_idx.at[j]], csems[b], add=True)

            bp = (b - LEAD) % NBUF

            @pl.when(jnp.logical_and(j >= LEAD, j + LEAD < nblk))
            def _():
                jp = j - LEAD
                pltpu.make_async_copy(
                    row_bufs[bp], acc_sh.at[dst_idx.at[jp]], ssems[bp]).wait()
                if cnt_sh is not None:
                    pltpu.make_async_copy(
                        ones_buf, cnt_sh.at[dst_idx.at[jp]], csems[bp]).wait()
                pltpu.async_copy(
                    table_sh.at[src_idx.at[j + LEAD]], row_bufs[bp], gsems[bp])
        return carry
    lax.fori_loop(0, nblk // NBUF, group, 0)

    for j in range(nblk - NBUF, nblk):
        b = j % NBUF
        pltpu.make_async_copy(
            row_bufs[b], acc_sh.at[dst_idx.at[j]], ssems[b]).wait()
        if cnt_sh is not None:
            pltpu.make_async_copy(
                ones_buf, cnt_sh.at[dst_idx.at[j]], csems[b]).wait()


def _zero_stage(stage):
    def zrow(i, carry):
        stage[i, :] = jnp.zeros((16,), jnp.float32)
        return carry
    lax.fori_loop(0, RPT, zrow, 0)


def _sage_sc_body(xp, xr, src1, dst1, src2, dst2,
                  acc2_out, cnt_out, h_out,
                  stage, xr_t, row_bufs, src_idx, dst_idx, ones_buf, zc,
                  table_sh, acc_sh, cnt_sh, sems):
    cid = lax.axis_index("c")
    sid = lax.axis_index("s")
    wid = sid * NC + cid
    gsems, ssems, csems = sems

    pltpu.sync_copy(xp.at[pl.ds(sid * RPT, RPT)], stage)
    pltpu.sync_copy(stage, table_sh.at[pl.ds(sid * RPT, RPT)])
    _zero_stage(stage)
    pltpu.sync_copy(stage, acc_sh.at[pl.ds(sid * RPT, RPT)])

    def zrow1(i, carry):
        zc[pl.ds(i * 16, 16)] = jnp.zeros((16,), jnp.float32)
        return carry
    lax.fori_loop(0, RPT // 16, zrow1, 0)
    pltpu.sync_copy(zc, cnt_sh.at[pl.ds(sid * RPT, RPT)])

    def orow(i, carry):
        ones_buf[pl.ds(i * 16, 16)] = jnp.ones((16,), jnp.float32)
        return carry
    lax.fori_loop(0, BLK // 16, orow, 0)
    plsc.subcore_barrier()

    pltpu.sync_copy(src1.at[sid], src_idx)
    pltpu.sync_copy(dst1.at[sid], dst_idx)
    _ring_loop(table_sh, acc_sh, cnt_sh, src_idx, dst_idx, NBLK1,
               row_bufs, ones_buf, gsems, ssems, csems)
    plsc.subcore_barrier()

    pltpu.sync_copy(acc_sh.at[pl.ds(sid * RPT, RPT)], stage)
    pltpu.sync_copy(cnt_sh.at[pl.ds(sid * RPT, RPT)], zc)
    pltpu.sync_copy(xr.at[pl.ds(sid * RPT, RPT)], xr_t)

    def hrow(g, carry):
        cv = zc[pl.ds(g * 16, 16)]
        for k in range(16):
            i = g * 16 + k
            c = jnp.maximum(cv[k], 1.0)
            stage[i, :] = jnp.maximum(stage[i, :] / c + xr_t[i, :], 0.0)
        return carry
    lax.fori_loop(0, RPT // 16, hrow, 0)
    pltpu.sync_copy(stage, table_sh.at[pl.ds(sid * RPT, RPT)])

    @pl.when(cid == 0)
    def _():
        pltpu.sync_copy(stage, h_out.at[pl.ds(sid * RPT, RPT)])

    pltpu.sync_copy(cnt_sh.at[pl.ds(sid * RPT, RPT)], zc)
    pltpu.sync_copy(zc, cnt_out.at[cid, pl.ds(sid * RPT, RPT)])
    _zero_stage(stage)
    pltpu.sync_copy(stage, acc_sh.at[pl.ds(sid * RPT, RPT)])
    plsc.subcore_barrier()

    pltpu.sync_copy(src2.at[wid], src_idx.at[pl.ds(0, NBLK2)])
    pltpu.sync_copy(dst2.at[wid], dst_idx.at[pl.ds(0, NBLK2)])
    _ring_loop(table_sh, acc_sh, None, src_idx, dst_idx, NBLK2,
               row_bufs, ones_buf, gsems, ssems, csems)
    plsc.subcore_barrier()

    pltpu.sync_copy(acc_sh.at[pl.ds(sid * RPT, RPT)], stage)
    pltpu.sync_copy(stage, acc2_out.at[cid, pl.ds(sid * RPT, RPT)])


_sage_sc = pl.kernel(
    _sage_sc_body,
    out_type=(
        jax.ShapeDtypeStruct((NC, NPAD, H), jnp.float32),
        jax.ShapeDtypeStruct((NC, NPAD), jnp.float32),
        jax.ShapeDtypeStruct((NPAD, H), jnp.float32),
    ),
    mesh=_mesh,
    scratch_types=(
        pltpu.VMEM((RPT, H), jnp.float32),
        pltpu.VMEM((RPT, H), jnp.float32),
        [pltpu.VMEM((BLK, H), jnp.float32) for _ in range(NBUF)],
        pltpu.VMEM((NBLK1, BLK), jnp.int32),
        pltpu.VMEM((NBLK1, BLK), jnp.int32),
        pltpu.VMEM((BLK,), jnp.float32),
        pltpu.VMEM((RPT,), jnp.float32),
        pltpu.VMEM_SHARED((NPAD, H), jnp.float32),
        pltpu.VMEM_SHARED((NPAD, H), jnp.float32),
        pltpu.VMEM_SHARED((NPAD,), jnp.float32),
        [[pltpu.SemaphoreType.DMA for _ in range(NBUF)] for _ in range(3)],
    ),
    compiler_params=pltpu.CompilerParams(use_tc_tiling_on_sc=False),
)

_ROWS = 400
_GRID = N // _ROWS


def _proj_body(x_ref, wl_ref, wr_ref, b_ref, xp_ref, xr_ref):
    xb = x_ref[...]
    dn = (((1,), (1,)), ((), ()))
    xp_ref[...] = lax.dot_general(xb, wl_ref[...], dn,
                                  preferred_element_type=jnp.float32)
    xr_ref[...] = lax.dot_general(xb, wr_ref[...], dn,
                                  preferred_element_type=jnp.float32) + b_ref[...]


def _out_body(acc_ref, cnt_ref, h_ref, wl_ref, wr_ref, b_ref, o_ref):
    a = acc_ref[0] + acc_ref[1]
    c = jnp.maximum(cnt_ref[...], 1.0)
    m = a / c
    dn = (((1,), (1,)), ((), ()))
    o_ref[...] = (lax.dot_general(m, wl_ref[...], dn,
                                  preferred_element_type=jnp.float32)
                  + lax.dot_general(h_ref[...], wr_ref[...], dn,
                                    preferred_element_type=jnp.float32)
                  + b_ref[...])


def kernel(x, edge_index, W1l, b1, W1r, W2l, b2, W2r):
    pad1 = NS * NBLK1 * BLK - E
    src1 = jnp.concatenate(
        [edge_index[0].reshape(NS, E // NS),
         jnp.zeros((NS, pad1 // NS), jnp.int32)], axis=1).reshape(NS, NBLK1, BLK)
    dst1 = jnp.concatenate(
        [edge_index[1].reshape(NS, E // NS),
         jnp.full((NS, pad1 // NS), DUMMY, jnp.int32)], axis=1).reshape(NS, NBLK1, BLK)
    pad2 = NW * NBLK2 * BLK - E
    src2 = jnp.concatenate(
        [edge_index[0].reshape(NW, E // NW),
         jnp.zeros((NW, pad2 // NW), jnp.int32)], axis=1).reshape(NW, NBLK2, BLK)
    dst2 = jnp.concatenate(
        [edge_index[1].reshape(NW, E // NW),
         jnp.full((NW, pad2 // NW), DUMMY, jnp.int32)], axis=1).reshape(NW, NBLK2, BLK)

    xp, xr = pl.pallas_call(
        _proj_body,
        grid=(_GRID,),
        in_specs=[
            pl.BlockSpec((_ROWS, D), lambda i: (i, 0)),
            pl.BlockSpec((H, D), lambda i: (0, 0)),
            pl.BlockSpec((H, D), lambda i: (0, 0)),
            pl.BlockSpec((1, H), lambda i: (0, 0)),
        ],
        out_specs=[
            pl.BlockSpec((_ROWS, H), lambda i: (i, 0)),
            pl.BlockSpec((_ROWS, H), lambda i: (i, 0)),
        ],
        out_shape=[
            jax.ShapeDtypeStruct((NPAD, H), jnp.float32),
            jax.ShapeDtypeStruct((NPAD, H), jnp.float32),
        ],
    )(x, W1l, W1r, b1[None, :])

    acc2, cnt, h = _sage_sc(xp, xr, src1, dst1, src2, dst2)
    cnt0 = cnt[0][:, None]

    out = pl.pallas_call(
        _out_body,
        grid=(_GRID,),
        in_specs=[
            pl.BlockSpec((NC, _ROWS, H), lambda i: (0, i, 0)),
            pl.BlockSpec((_ROWS, 1), lambda i: (i, 0)),
            pl.BlockSpec((_ROWS, H), lambda i: (i, 0)),
            pl.BlockSpec((C, H), lambda i: (0, 0)),
            pl.BlockSpec((C, H), lambda i: (0, 0)),
            pl.BlockSpec((1, C), lambda i: (0, 0)),
        ],
        out_specs=pl.BlockSpec((_ROWS, C), lambda i: (i, 0)),
        out_shape=jax.ShapeDtypeStruct((N, C), jnp.float32),
    )(acc2, cnt0, h, W2l, W2r, b2[None, :])

    return out


# speedup vs baseline: 21.7206x; 1.0465x over previous
import jax
import jax.numpy as jnp
from jax import lax
from jax.experimental import pallas as pl
from jax.experimental.pallas import tpu as pltpu
from jax.experimental.pallas import tpu_sc as plsc

N = 10000
E = 320000
D = 128
H = 16
C = 47

NC = 2
NS = 16
NW = NC * NS
BLK = 128
NBUF = 8
LEAD = 4
NBLK1 = 160
NBLK2 = 80
NPAD = 10240
DUMMY = NPAD - 2
RPT = NPAD // NS
TPT = N // NS

_mesh = plsc.VectorSubcoreMesh(core_axis_name="c", subcore_axis_name="s")


def _ring_loop(table_sh, acc_sh, cnt_sh, src_idx, dst_idx, nblk,
               row_bufs, ones_buf, gsems, ssems, csems):
    for b in range(NBUF):
        pltpu.async_copy(table_sh.at[src_idx.at[b]], row_bufs[b], gsems[b])

    def group(g, carry):
        for b in range(NBUF):
            j = g * NBUF + b
            pltpu.make_async_copy(
                table_sh.at[src_idx.at[j]], row_bufs[b], gsems[b]).wait()
            pltpu.async_copy(
                row_bufs[b], acc_sh.at[dst_idx.at[j]], ssems[b], add=True)
            if cnt_sh is not None:
                pltpu.async_copy(
                    ones_buf, cnt_sh.at[dst_idx.at[j]], csems[b], add=True)

            bp = (b - LEAD) % NBUF

            @pl.when(jnp.logical_and(j >= LEAD, j + LEAD < nblk))
            def _():
                jp = j - LEAD
                pltpu.make_async_copy(
                    row_bufs[bp], acc_sh.at[dst_idx.at[jp]], ssems[bp]).wait()
                if cnt_sh is not None:
                    pltpu.make_async_copy(
                        ones_buf, cnt_sh.at[dst_idx.at[jp]], csems[bp]).wait()
                pltpu.async_copy(
                    table_sh.at[src_idx.at[j + LEAD]], row_bufs[bp], gsems[bp])
        return carry
    lax.fori_loop(0, nblk // NBUF, group, 0)

    for j in range(nblk - NBUF, nblk):
        b = j % NBUF
        pltpu.make_async_copy(
            row_bufs[b], acc_sh.at[dst_idx.at[j]], ssems[b]).wait()
        if cnt_sh is not None:
            pltpu.make_async_copy(
                ones_buf, cnt_sh.at[dst_idx.at[j]], csems[b]).wait()


def _zero_stage(stage):
    def zrow(i, carry):
        stage[i, :] = jnp.zeros((16,), jnp.float32)
        return carry
    lax.fori_loop(0, RPT, zrow, 0)


def _sage_sc_body(xp, xr, src1, dst1, src2, dst2,
                  acc2_out, cnt_out, h_out,
                  stage, xr_t, row_bufs, src_idx, dst_idx, ones_buf, zc,
                  table_sh, acc_sh, cnt_sh, sems):
    cid = lax.axis_index("c")
    sid = lax.axis_index("s")
    wid = sid * NC + cid
    gsems, ssems, csems = sems

    pltpu.sync_copy(xp.at[pl.ds(sid * RPT, RPT)], stage)
    pltpu.sync_copy(stage, table_sh.at[pl.ds(sid * RPT, RPT)])
    _zero_stage(stage)
    pltpu.sync_copy(stage, acc_sh.at[pl.ds(sid * RPT, RPT)])

    def zrow1(i, carry):
        zc[pl.ds(i * 16, 16)] = jnp.zeros((16,), jnp.float32)
        return carry
    lax.fori_loop(0, RPT // 16, zrow1, 0)
    pltpu.sync_copy(zc, cnt_sh.at[pl.ds(sid * RPT, RPT)])

    def orow(i, carry):
        ones_buf[pl.ds(i * 16, 16)] = jnp.ones((16,), jnp.float32)
        return carry
    lax.fori_loop(0, BLK // 16, orow, 0)
    plsc.subcore_barrier()

    pltpu.sync_copy(src1.at[sid], src_idx)
    pltpu.sync_copy(dst1.at[sid], dst_idx)
    _ring_loop(table_sh, acc_sh, cnt_sh, src_idx, dst_idx, NBLK1,
               row_bufs, ones_buf, gsems, ssems, csems)
    plsc.subcore_barrier()

    pltpu.sync_copy(acc_sh.at[pl.ds(sid * RPT, RPT)], stage)
    pltpu.sync_copy(cnt_sh.at[pl.ds(sid * RPT, RPT)], zc)
    pltpu.sync_copy(xr.at[pl.ds(sid * RPT, RPT)], xr_t)

    def hrow(g, carry):
        cv = zc[pl.ds(g * 16, 16)]
        for k in range(16):
            i = g * 16 + k
            c = jnp.maximum(cv[k], 1.0)
            stage[i, :] = jnp.maximum(stage[i, :] / c + xr_t[i, :], 0.0)
        return carry
    lax.fori_loop(0, RPT // 16, hrow, 0)
    pltpu.sync_copy(stage, table_sh.at[pl.ds(sid * RPT, RPT)])

    @pl.when(cid == 0)
    def _():
        pltpu.sync_copy(stage, h_out.at[pl.ds(sid * RPT, RPT)])

    pltpu.sync_copy(cnt_sh.at[pl.ds(sid * RPT, RPT)], zc)
    pltpu.sync_copy(zc, cnt_out.at[cid, pl.ds(sid * RPT, RPT)])
    _zero_stage(stage)
    pltpu.sync_copy(stage, acc_sh.at[pl.ds(sid * RPT, RPT)])
    plsc.subcore_barrier()

    pltpu.sync_copy(src2.at[wid], src_idx.at[pl.ds(0, NBLK2)])
    pltpu.sync_copy(dst2.at[wid], dst_idx.at[pl.ds(0, NBLK2)])
    _ring_loop(table_sh, acc_sh, None, src_idx, dst_idx, NBLK2,
               row_bufs, ones_buf, gsems, ssems, csems)
    plsc.subcore_barrier()

    pltpu.sync_copy(acc_sh.at[pl.ds(sid * RPT, RPT)], stage)
    pltpu.sync_copy(stage, acc2_out.at[cid, pl.ds(sid * RPT, RPT)])


_sage_sc = pl.kernel(
    _sage_sc_body,
    out_type=(
        jax.ShapeDtypeStruct((NC, NPAD, H), jnp.float32),
        jax.ShapeDtypeStruct((NC, NPAD), jnp.float32),
        jax.ShapeDtypeStruct((NPAD, H), jnp.float32),
    ),
    mesh=_mesh,
    scratch_types=(
        pltpu.VMEM((RPT, H), jnp.float32),
        pltpu.VMEM((RPT, H), jnp.float32),
        [pltpu.VMEM((BLK, H), jnp.float32) for _ in range(NBUF)],
        pltpu.VMEM((NBLK1, BLK), jnp.int32),
        pltpu.VMEM((NBLK1, BLK), jnp.int32),
        pltpu.VMEM((BLK,), jnp.float32),
        pltpu.VMEM((RPT,), jnp.float32),
        pltpu.VMEM_SHARED((NPAD, H), jnp.float32),
        pltpu.VMEM_SHARED((NPAD, H), jnp.float32),
        pltpu.VMEM_SHARED((NPAD,), jnp.float32),
        [[pltpu.SemaphoreType.DMA for _ in range(NBUF)] for _ in range(3)],
    ),
    compiler_params=pltpu.CompilerParams(use_tc_tiling_on_sc=False),
)

_ROWS = 400
_GRID = N // _ROWS


def _proj_body(x_ref, wl_ref, wr_ref, b_ref, xp_ref, xr_ref):
    xb = x_ref[...]
    dn = (((1,), (1,)), ((), ()))
    xp_ref[...] = lax.dot_general(xb, wl_ref[...], dn,
                                  preferred_element_type=jnp.float32)
    xr_ref[...] = lax.dot_general(xb, wr_ref[...], dn,
                                  preferred_element_type=jnp.float32) + b_ref[...]


def _out_body(acc_ref, cnt_ref, h_ref, wl_ref, wr_ref, b_ref, o_ref):
    a = acc_ref[0] + acc_ref[1]
    c = jnp.maximum(cnt_ref[...], 1.0)
    m = a / c
    dn = (((1,), (1,)), ((), ()))
    o_ref[...] = (lax.dot_general(m, wl_ref[...], dn,
                                  preferred_element_type=jnp.float32)
                  + lax.dot_general(h_ref[...], wr_ref[...], dn,
                                    preferred_element_type=jnp.float32)
                  + b_ref[...])


def kernel(x, edge_index, W1l, b1, W1r, W2l, b2, W2r):
    def _pad_edges(e, nchunk, nblk, fill_dst):
        per = nchunk * nblk * BLK // nchunk
        padw = per - E // nchunk
        if fill_dst:
            fill = N + (jnp.arange(padw, dtype=jnp.int32) % (NPAD - N))
        else:
            fill = jnp.arange(padw, dtype=jnp.int32) % 128
        return jnp.concatenate(
            [e.reshape(nchunk, E // nchunk),
             jnp.broadcast_to(fill, (nchunk, padw))], axis=1
        ).reshape(nchunk, nblk, BLK)

    src1 = _pad_edges(edge_index[0], NS, NBLK1, False)
    dst1 = _pad_edges(edge_index[1], NS, NBLK1, True)
    src2 = _pad_edges(edge_index[0], NW, NBLK2, False)
    dst2 = _pad_edges(edge_index[1], NW, NBLK2, True)

    xp, xr = pl.pallas_call(
        _proj_body,
        grid=(_GRID,),
        in_specs=[
            pl.BlockSpec((_ROWS, D), lambda i: (i, 0)),
            pl.BlockSpec((H, D), lambda i: (0, 0)),
            pl.BlockSpec((H, D), lambda i: (0, 0)),
            pl.BlockSpec((1, H), lambda i: (0, 0)),
        ],
        out_specs=[
            pl.BlockSpec((_ROWS, H), lambda i: (i, 0)),
            pl.BlockSpec((_ROWS, H), lambda i: (i, 0)),
        ],
        out_shape=[
            jax.ShapeDtypeStruct((NPAD, H), jnp.float32),
            jax.ShapeDtypeStruct((NPAD, H), jnp.float32),
        ],
    )(x, W1l, W1r, b1[None, :])

    acc2, cnt, h = _sage_sc(xp, xr, src1, dst1, src2, dst2)
    cnt0 = cnt[0][:, None]

    out = pl.pallas_call(
        _out_body,
        grid=(_GRID,),
        in_specs=[
            pl.BlockSpec((NC, _ROWS, H), lambda i: (0, i, 0)),
            pl.BlockSpec((_ROWS, 1), lambda i: (i, 0)),
            pl.BlockSpec((_ROWS, H), lambda i: (i, 0)),
            pl.BlockSpec((C, H), lambda i: (0, 0)),
            pl.BlockSpec((C, H), lambda i: (0, 0)),
            pl.BlockSpec((1, C), lambda i: (0, 0)),
        ],
        out_specs=pl.BlockSpec((_ROWS, C), lambda i: (i, 0)),
        out_shape=jax.ShapeDtypeStruct((N, C), jnp.float32),
    )(acc2, cnt0, h, W2l, W2r, b2[None, :])

    return out
